# baseline (speedup 1.0000x reference)
import functools

import jax
import jax.numpy as jnp
from jax import lax
from jax.experimental import pallas as pl
from jax.experimental.pallas import tpu as pltpu, tpu_sc as plsc

N = 10000
E = 320000
D = 128
DE = 16

NC = 2
NS = 16
NW = NC * NS
EW = E // NW
CE = 80
NCHUNK = EW // CE
NPAIR = (NCHUNK + 1) // 2
EW2 = (E // 2) // NW
CE2 = 40
NCHUNK2 = EW2 // CE2
NPAIR2 = (NCHUNK2 + 1) // 2
N_PAD = 10240
ROWS_PER_TILE = N_PAD // NS

_DOT_T = (((1,), (1,)), ((), ()))


def _node_body(x_ref, wlin_ref, blin_ref, wp0_ref, wp1_ref, btot_ref,
               wp2_ref, we3_ref, be3_ref, gamma_ref, beta_ref,
               y_ref, baseh_ref, wc_ref, c2_ref):
    x2 = lax.dot_general(x_ref[...], wlin_ref[...], _DOT_T,
                         preferred_element_type=jnp.float32) + blin_ref[...]
    y_ref[...] = lax.dot_general(x2, wp1_ref[...], _DOT_T,
                                 preferred_element_type=jnp.float32)
    baseh_ref[...] = 0.5 * (
        lax.dot_general(x2, wp0_ref[...], _DOT_T,
                        preferred_element_type=jnp.float32) + btot_ref[...])
    wc0 = lax.dot_general(wp2_ref[...], we3_ref[...],
                          (((1,), (0,)), ((), ())),
                          preferred_element_type=jnp.float32)
    wc_ref[...] = wc0 * gamma_ref[...]
    c2_ref[...] = (lax.dot_general(be3_ref[...], wp2_ref[...], _DOT_T,
                                   preferred_element_type=jnp.float32)
                   + lax.dot_general(beta_ref[...], wc0, _DOT_T,
                                     preferred_element_type=jnp.float32))


SB = 80
EB = 4000


def _edge_body(seg, step0, ea_hbm, we0_ref, be0_ref, wc_ref, c2_ref, z_ref,
               ea_v, sem):
    i = pl.program_id(0)
    pltpu.make_async_copy(
        ea_hbm.at[pl.ds((step0 + i) * EB, EB)], ea_v, sem).start()
    pltpu.make_async_copy(
        ea_hbm.at[pl.ds((step0 + i) * EB, EB)], ea_v, sem).wait()
    a_all = lax.dot_general(ea_v[...], we0_ref[...], _DOT_T,
                            preferred_element_type=jnp.float32)
    for s in range(EB // SB):
        a = a_all[s * SB:(s + 1) * SB, :] + be0_ref[...]
        e0 = jnp.maximum(a, 0.0)
        mu = jnp.mean(e0, axis=-1, keepdims=True)
        m2 = jnp.mean(e0 * e0, axis=-1, keepdims=True)
        rs = lax.rsqrt(m2 - mu * mu + 1e-5)
        nh = (e0 - mu) * rs
        z_ref[s * SB:(s + 1) * SB, :] = lax.dot_general(
            nh.astype(jnp.bfloat16), wc_ref[...], _DOT_T,
            preferred_element_type=jnp.float32) + c2_ref[...]


def _sc_gather_body(y_hbm, ei_hbm, baseh_hbm, out_hbm,
                    src_v, dst_v, row_v, acc, sems, dsems):
    cid = lax.axis_index("c")
    sid = lax.axis_index("s")
    wid = cid * NS + sid
    row0 = sid * ROWS_PER_TILE
    pltpu.sync_copy(baseh_hbm.at[pl.ds(row0, ROWS_PER_TILE)],
                    acc.at[pl.ds(row0, ROWS_PER_TILE)])
    edge0 = wid * EW
    pltpu.sync_copy(ei_hbm.at[pl.ds(edge0, EW)], src_v)
    plsc.subcore_barrier()

    pltpu.async_copy(ei_hbm.at[pl.ds(E + edge0, CE)], dst_v.at[0], dsems[0])
    pltpu.async_copy(y_hbm.at[src_v.at[pl.ds(0, CE)]], row_v.at[0], sems[0])

    def pair(p, carry):
        for b in range(2):
            k = 2 * p + b
            nb = 1 - b

            @pl.when(k < NCHUNK)
            def _():
                @pl.when(k + 1 < NCHUNK)
                def _():
                    pltpu.async_copy(
                        ei_hbm.at[pl.ds(E + edge0 + (k + 1) * CE, CE)],
                        dst_v.at[nb], dsems[nb])
                    pltpu.async_copy(
                        y_hbm.at[src_v.at[pl.ds((k + 1) * CE, CE)]],
                        row_v.at[nb], sems[nb])
                pltpu.make_async_copy(y_hbm.at[src_v.at[pl.ds(k * CE, CE)]],
                                      row_v.at[b], sems[b]).wait()
                pltpu.make_async_copy(
                    ei_hbm.at[pl.ds(E + edge0 + k * CE, CE)],
                    dst_v.at[b], dsems[b]).wait()
                pltpu.sync_copy(row_v.at[b], acc.at[dst_v.at[b]], add=True)
        return carry

    lax.fori_loop(0, NPAIR, pair, 0)
    plsc.subcore_barrier()
    pltpu.sync_copy(acc.at[pl.ds(row0, ROWS_PER_TILE)],
                    out_hbm.at[cid, pl.ds(row0, ROWS_PER_TILE)])


def _sc_scatter_body(seg, z_hbm, ei_hbm, part_hbm, out_hbm,
                     dst_v, row_v, acc, sems, dsems):
    cid = lax.axis_index("c")
    sid = lax.axis_index("s")
    wid = cid * NS + sid
    row0 = sid * ROWS_PER_TILE
    pltpu.sync_copy(part_hbm.at[cid, pl.ds(row0, ROWS_PER_TILE)],
                    acc.at[pl.ds(row0, ROWS_PER_TILE)])
    plsc.subcore_barrier()

    zrow0 = wid * EW2
    idx0 = E + seg * (E // 2) + zrow0

    pltpu.async_copy(ei_hbm.at[pl.ds(idx0, CE2)], dst_v.at[0], dsems[0])
    pltpu.async_copy(z_hbm.at[pl.ds(zrow0, CE2)], row_v.at[0], sems[0])

    def pair(p, carry):
        for b in range(2):
            k = 2 * p + b
            nb = 1 - b

            @pl.when(k < NCHUNK2)
            def _():
                @pl.when(k + 1 < NCHUNK2)
                def _():
                    pltpu.async_copy(
                        ei_hbm.at[pl.ds(idx0 + (k + 1) * CE2, CE2)],
                        dst_v.at[nb], dsems[nb])
                    pltpu.async_copy(
                        z_hbm.at[pl.ds(zrow0 + (k + 1) * CE2, CE2)],
                        row_v.at[nb], sems[nb])
                pltpu.make_async_copy(z_hbm.at[pl.ds(zrow0 + k * CE2, CE2)],
                                      row_v.at[b], sems[b]).wait()
                pltpu.make_async_copy(
                    ei_hbm.at[pl.ds(idx0 + k * CE2, CE2)],
                    dst_v.at[b], dsems[b]).wait()
                pltpu.sync_copy(row_v.at[b], acc.at[dst_v.at[b]], add=True)
        return carry

    lax.fori_loop(0, NPAIR2, pair, 0)
    plsc.subcore_barrier()
    pltpu.sync_copy(acc.at[pl.ds(row0, ROWS_PER_TILE)],
                    out_hbm.at[cid, pl.ds(row0, ROWS_PER_TILE)])


def _combine_body(a_ref, b_ref, o_ref):
    o_ref[...] = a_ref[...] + b_ref[...]


def kernel(x, edge_index, edge_attr, W_lin, b_lin, W_e0, b_e0, ln_gamma,
           ln_beta, W_e3, b_e3, W_proj, b_proj, bias_p):
    ei = edge_index.astype(jnp.int32).reshape(2 * E)
    Wp0 = W_proj[:, 0:D]
    Wp1 = W_proj[:, D:2 * D]
    Wp2 = W_proj[:, 2 * D:3 * D]
    b_tot = (b_proj + bias_p).reshape(1, D)

    y, base_half, Wc, c2 = pl.pallas_call(
        _node_body,
        out_shape=(
            jax.ShapeDtypeStruct((N, D), jnp.float32),
            jax.ShapeDtypeStruct((N, D), jnp.float32),
            jax.ShapeDtypeStruct((D, D), jnp.float32),
            jax.ShapeDtypeStruct((1, D), jnp.float32),
        ),
    )(x, W_lin, b_lin.reshape(1, D), Wp0, Wp1, b_tot, Wp2, W_e3,
      b_e3.reshape(1, D), ln_gamma.reshape(1, D), ln_beta.reshape(1, D))

    NSTEP_HALF = E // 2 // EB
    Wc_bf = Wc.astype(jnp.bfloat16)

    def edge_half(seg):
        return pl.pallas_call(
            functools.partial(_edge_body, seg, seg * NSTEP_HALF),
            grid=(NSTEP_HALF,),
            in_specs=[
                pl.BlockSpec(memory_space=pltpu.MemorySpace.HBM),
                pl.BlockSpec((D, DE), lambda i: (0, 0)),
                pl.BlockSpec((1, D), lambda i: (0, 0)),
                pl.BlockSpec((D, D), lambda i: (0, 0)),
                pl.BlockSpec((1, D), lambda i: (0, 0)),
            ],
            out_specs=pl.BlockSpec((EB, D), lambda i: (i, 0)),
            out_shape=jax.ShapeDtypeStruct((E // 2, D), jnp.float32),
            scratch_shapes=[
                pltpu.VMEM((EB, DE), jnp.float32),
                pltpu.SemaphoreType.DMA,
            ],
        )(edge_attr, W_e0, b_e0.reshape(1, D), Wc_bf, c2)

    z_a = edge_half(0)
    z_b = edge_half(1)

    base_half_pad = jnp.pad(base_half, ((0, N_PAD - N), (0, 0)))
    mesh = plsc.VectorSubcoreMesh(core_axis_name="c", subcore_axis_name="s",
                                  num_cores=NC, num_subcores=NS)
    part1 = pl.kernel(
        _sc_gather_body,
        out_type=jax.ShapeDtypeStruct((NC, N_PAD, D), jnp.float32),
        mesh=mesh,
        scratch_types=[
            pltpu.VMEM((EW,), jnp.int32),
            pltpu.VMEM((2, CE), jnp.int32),
            pltpu.VMEM((2, CE, D), jnp.float32),
            pltpu.VMEM_SHARED((N_PAD, D), jnp.float32),
            (pltpu.SemaphoreType.DMA, pltpu.SemaphoreType.DMA),
            (pltpu.SemaphoreType.DMA, pltpu.SemaphoreType.DMA),
        ],
    )(y, ei, base_half_pad)

    def scatter_half(seg, zseg, part):
        return pl.kernel(
            functools.partial(_sc_scatter_body, seg),
            out_type=jax.ShapeDtypeStruct((NC, N_PAD, D), jnp.float32),
            mesh=mesh,
            scratch_types=[
                pltpu.VMEM((2, CE2), jnp.int32),
                pltpu.VMEM((2, CE2, D), jnp.float32),
                pltpu.VMEM_SHARED((N_PAD, D), jnp.float32),
                (pltpu.SemaphoreType.DMA, pltpu.SemaphoreType.DMA),
                (pltpu.SemaphoreType.DMA, pltpu.SemaphoreType.DMA),
            ],
        )(zseg, ei, part)

    part2 = scatter_half(0, z_a, part1)
    partials = scatter_half(1, z_b, part2)

    NB = 1000
    h = pl.pallas_call(
        _combine_body,
        grid=(N // NB,),
        in_specs=[
            pl.BlockSpec((NB, D), lambda i: (i, 0)),
            pl.BlockSpec((NB, D), lambda i: (i, 0)),
        ],
        out_specs=pl.BlockSpec((NB, D), lambda i: (i, 0)),
        out_shape=jax.ShapeDtypeStruct((N, D), jnp.float32),
    )(partials[0], partials[1])
    return h


# speedup vs baseline: 1.3015x; 1.3015x over previous
import functools

import jax
import jax.numpy as jnp
from jax import lax
from jax.experimental import pallas as pl
from jax.experimental.pallas import tpu as pltpu, tpu_sc as plsc

N = 10000
E = 320000
D = 128
DE = 16

NC = 2
NS = 16
NW = NC * NS
EW = E // NW
CE = 80
NCHUNK = EW // CE
NPAIR = (NCHUNK + 1) // 2
EW2 = (E // 2) // NW
CE2 = 40
NCHUNK2 = EW2 // CE2
NPAIR2 = (NCHUNK2 + 1) // 2
N_PAD = 10240
ROWS_PER_TILE = N_PAD // NS

_DOT_T = (((1,), (1,)), ((), ()))


def _node_body(x_ref, wlin_ref, blin_ref, wp0_ref, wp1_ref, btot_ref,
               wp2_ref, we3_ref, be3_ref, gamma_ref, beta_ref,
               y_ref, baseh_ref, wc_ref, c2_ref):
    x2 = lax.dot_general(x_ref[...], wlin_ref[...], _DOT_T,
                         preferred_element_type=jnp.float32) + blin_ref[...]
    y_ref[...] = lax.dot_general(x2, wp1_ref[...], _DOT_T,
                                 preferred_element_type=jnp.float32)
    baseh_ref[...] = 0.5 * (
        lax.dot_general(x2, wp0_ref[...], _DOT_T,
                        preferred_element_type=jnp.float32) + btot_ref[...])
    wc0 = lax.dot_general(wp2_ref[...], we3_ref[...],
                          (((1,), (0,)), ((), ())),
                          preferred_element_type=jnp.float32)
    wc_ref[...] = wc0 * gamma_ref[...]
    c2_ref[...] = (lax.dot_general(be3_ref[...], wp2_ref[...], _DOT_T,
                                   preferred_element_type=jnp.float32)
                   + lax.dot_general(beta_ref[...], wc0, _DOT_T,
                                     preferred_element_type=jnp.float32))


SB = 80
EB = 4000


def _edge_body(seg, step0, nstep, ea_hbm, we0_ref, be0_ref, wc_ref, c2_ref,
               z_ref, ea_v, sems):
    i = pl.program_id(0)

    def blk_copy(step, slot):
        return pltpu.make_async_copy(
            ea_hbm.at[pl.ds((step0 + step) * EB, EB)], ea_v.at[slot],
            sems.at[slot])

    @pl.when(i == 0)
    def _():
        blk_copy(0, 0).start()

    @pl.when(i + 1 < nstep)
    def _():
        blk_copy(i + 1, (i + 1) % 2).start()
    blk_copy(i, i % 2).wait()
    a_all = lax.dot_general(ea_v[i % 2], we0_ref[...], _DOT_T,
                            preferred_element_type=jnp.float32)
    for s in range(EB // SB):
        a = a_all[s * SB:(s + 1) * SB, :] + be0_ref[...]
        e0 = jnp.maximum(a, 0.0)
        mu = jnp.mean(e0, axis=-1, keepdims=True)
        m2 = jnp.mean(e0 * e0, axis=-1, keepdims=True)
        rs = lax.rsqrt(m2 - mu * mu + 1e-5)
        nh = (e0 - mu) * rs
        z_ref[s * SB:(s + 1) * SB, :] = lax.dot_general(
            nh.astype(jnp.bfloat16), wc_ref[...], _DOT_T,
            preferred_element_type=jnp.float32) + c2_ref[...]


def _sc_gather_body(y_hbm, ei_hbm, baseh_hbm, out_hbm,
                    src_v, dst_v, row_v, acc, sems, dsems):
    cid = lax.axis_index("c")
    sid = lax.axis_index("s")
    wid = cid * NS + sid
    row0 = sid * ROWS_PER_TILE
    pltpu.sync_copy(baseh_hbm.at[pl.ds(row0, ROWS_PER_TILE)],
                    acc.at[pl.ds(row0, ROWS_PER_TILE)])
    edge0 = wid * EW
    pltpu.sync_copy(ei_hbm.at[pl.ds(edge0, EW)], src_v)
    plsc.subcore_barrier()

    pltpu.async_copy(ei_hbm.at[pl.ds(E + edge0, CE)], dst_v.at[0], dsems[0])
    pltpu.async_copy(y_hbm.at[src_v.at[pl.ds(0, CE)]], row_v.at[0], sems[0])

    def pair(p, carry):
        for b in range(2):
            k = 2 * p + b
            nb = 1 - b

            @pl.when(k < NCHUNK)
            def _():
                @pl.when(k + 1 < NCHUNK)
                def _():
                    pltpu.async_copy(
                        ei_hbm.at[pl.ds(E + edge0 + (k + 1) * CE, CE)],
                        dst_v.at[nb], dsems[nb])
                    pltpu.async_copy(
                        y_hbm.at[src_v.at[pl.ds((k + 1) * CE, CE)]],
                        row_v.at[nb], sems[nb])
                pltpu.make_async_copy(y_hbm.at[src_v.at[pl.ds(k * CE, CE)]],
                                      row_v.at[b], sems[b]).wait()
                pltpu.make_async_copy(
                    ei_hbm.at[pl.ds(E + edge0 + k * CE, CE)],
                    dst_v.at[b], dsems[b]).wait()
                pltpu.sync_copy(row_v.at[b], acc.at[dst_v.at[b]], add=True)
        return carry

    lax.fori_loop(0, NPAIR, pair, 0)
    plsc.subcore_barrier()
    pltpu.sync_copy(acc.at[pl.ds(row0, ROWS_PER_TILE)],
                    out_hbm.at[cid, pl.ds(row0, ROWS_PER_TILE)])


def _sc_scatter_body(seg, z_hbm, ei_hbm, part_hbm, out_hbm,
                     dst_v, row_v, acc, sems, dsems):
    cid = lax.axis_index("c")
    sid = lax.axis_index("s")
    wid = cid * NS + sid
    row0 = sid * ROWS_PER_TILE
    pltpu.sync_copy(part_hbm.at[cid, pl.ds(row0, ROWS_PER_TILE)],
                    acc.at[pl.ds(row0, ROWS_PER_TILE)])
    plsc.subcore_barrier()

    zrow0 = wid * EW2
    idx0 = E + seg * (E // 2) + zrow0

    pltpu.async_copy(ei_hbm.at[pl.ds(idx0, CE2)], dst_v.at[0], dsems[0])
    pltpu.async_copy(z_hbm.at[pl.ds(zrow0, CE2)], row_v.at[0], sems[0])

    def pair(p, carry):
        for b in range(2):
            k = 2 * p + b
            nb = 1 - b

            @pl.when(k < NCHUNK2)
            def _():
                @pl.when(k + 1 < NCHUNK2)
                def _():
                    pltpu.async_copy(
                        ei_hbm.at[pl.ds(idx0 + (k + 1) * CE2, CE2)],
                        dst_v.at[nb], dsems[nb])
                    pltpu.async_copy(
                        z_hbm.at[pl.ds(zrow0 + (k + 1) * CE2, CE2)],
                        row_v.at[nb], sems[nb])
                pltpu.make_async_copy(z_hbm.at[pl.ds(zrow0 + k * CE2, CE2)],
                                      row_v.at[b], sems[b]).wait()
                pltpu.make_async_copy(
                    ei_hbm.at[pl.ds(idx0 + k * CE2, CE2)],
                    dst_v.at[b], dsems[b]).wait()
                pltpu.sync_copy(row_v.at[b], acc.at[dst_v.at[b]], add=True)
        return carry

    lax.fori_loop(0, NPAIR2, pair, 0)
    plsc.subcore_barrier()
    pltpu.sync_copy(acc.at[pl.ds(row0, ROWS_PER_TILE)],
                    out_hbm.at[cid, pl.ds(row0, ROWS_PER_TILE)])


def _combine_body(a_ref, b_ref, o_ref):
    o_ref[...] = a_ref[...] + b_ref[...]


def kernel(x, edge_index, edge_attr, W_lin, b_lin, W_e0, b_e0, ln_gamma,
           ln_beta, W_e3, b_e3, W_proj, b_proj, bias_p):
    ei = edge_index.astype(jnp.int32).reshape(2 * E)
    Wp0 = W_proj[:, 0:D]
    Wp1 = W_proj[:, D:2 * D]
    Wp2 = W_proj[:, 2 * D:3 * D]
    b_tot = (b_proj + bias_p).reshape(1, D)

    y, base_half, Wc, c2 = pl.pallas_call(
        _node_body,
        out_shape=(
            jax.ShapeDtypeStruct((N, D), jnp.float32),
            jax.ShapeDtypeStruct((N, D), jnp.float32),
            jax.ShapeDtypeStruct((D, D), jnp.float32),
            jax.ShapeDtypeStruct((1, D), jnp.float32),
        ),
    )(x, W_lin, b_lin.reshape(1, D), Wp0, Wp1, b_tot, Wp2, W_e3,
      b_e3.reshape(1, D), ln_gamma.reshape(1, D), ln_beta.reshape(1, D))

    NSTEP_HALF = E // 2 // EB
    Wc_bf = Wc.astype(jnp.bfloat16)

    def edge_half(seg):
        return pl.pallas_call(
            functools.partial(_edge_body, seg, seg * NSTEP_HALF, NSTEP_HALF),
            grid=(NSTEP_HALF,),
            in_specs=[
                pl.BlockSpec(memory_space=pltpu.MemorySpace.HBM),
                pl.BlockSpec((D, DE), lambda i: (0, 0)),
                pl.BlockSpec((1, D), lambda i: (0, 0)),
                pl.BlockSpec((D, D), lambda i: (0, 0)),
                pl.BlockSpec((1, D), lambda i: (0, 0)),
            ],
            out_specs=pl.BlockSpec((EB, D), lambda i: (i, 0)),
            out_shape=jax.ShapeDtypeStruct((E // 2, D), jnp.float32),
            scratch_shapes=[
                pltpu.VMEM((2, EB, DE), jnp.float32),
                pltpu.SemaphoreType.DMA((2,)),
            ],
        )(edge_attr, W_e0, b_e0.reshape(1, D), Wc_bf, c2)

    z_a = edge_half(0)
    z_b = edge_half(1)

    base_half_pad = jnp.pad(base_half, ((0, N_PAD - N), (0, 0)))
    mesh = plsc.VectorSubcoreMesh(core_axis_name="c", subcore_axis_name="s",
                                  num_cores=NC, num_subcores=NS)
    part1 = pl.kernel(
        _sc_gather_body,
        out_type=jax.ShapeDtypeStruct((NC, N_PAD, D), jnp.float32),
        mesh=mesh,
        scratch_types=[
            pltpu.VMEM((EW,), jnp.int32),
            pltpu.VMEM((2, CE), jnp.int32),
            pltpu.VMEM((2, CE, D), jnp.float32),
            pltpu.VMEM_SHARED((N_PAD, D), jnp.float32),
            (pltpu.SemaphoreType.DMA, pltpu.SemaphoreType.DMA),
            (pltpu.SemaphoreType.DMA, pltpu.SemaphoreType.DMA),
        ],
    )(y, ei, base_half_pad)

    def scatter_half(seg, zseg, part):
        return pl.kernel(
            functools.partial(_sc_scatter_body, seg),
            out_type=jax.ShapeDtypeStruct((NC, N_PAD, D), jnp.float32),
            mesh=mesh,
            scratch_types=[
                pltpu.VMEM((2, CE2), jnp.int32),
                pltpu.VMEM((2, CE2, D), jnp.float32),
                pltpu.VMEM_SHARED((N_PAD, D), jnp.float32),
                (pltpu.SemaphoreType.DMA, pltpu.SemaphoreType.DMA),
                (pltpu.SemaphoreType.DMA, pltpu.SemaphoreType.DMA),
            ],
        )(zseg, ei, part)

    part2 = scatter_half(0, z_a, part1)
    partials = scatter_half(1, z_b, part2)

    NB = 1000
    h = pl.pallas_call(
        _combine_body,
        grid=(N // NB,),
        in_specs=[
            pl.BlockSpec((NB, D), lambda i: (i, 0)),
            pl.BlockSpec((NB, D), lambda i: (i, 0)),
        ],
        out_specs=pl.BlockSpec((NB, D), lambda i: (i, 0)),
        out_shape=jax.ShapeDtypeStruct((N, D), jnp.float32),
    )(partials[0], partials[1])
    return h


# speedup vs baseline: 1.4594x; 1.1214x over previous
import functools

import jax
import jax.numpy as jnp
from jax import lax
from jax.experimental import pallas as pl
from jax.experimental.pallas import tpu as pltpu, tpu_sc as plsc

N = 10000
E = 320000
D = 128
DE = 16

NC = 2
NS = 16
NW = NC * NS
EW = E // NW
CE = 80
NCHUNK = EW // CE
NPAIR = (NCHUNK + 1) // 2
EW2 = (E // 2) // NW
CE2 = 40
NCHUNK2 = EW2 // CE2
NPAIR2 = (NCHUNK2 + 1) // 2
N_PAD = 10240
ROWS_PER_TILE = N_PAD // NS

_DOT_T = (((1,), (1,)), ((), ()))


def _node_body(x_ref, wlin_ref, blin_ref, wp0_ref, wp1_ref, btot_ref,
               wp2_ref, we3_ref, be3_ref, gamma_ref, beta_ref,
               y_ref, baseh_ref, wc_ref, c2_ref):
    x2 = lax.dot_general(x_ref[...], wlin_ref[...], _DOT_T,
                         preferred_element_type=jnp.float32) + blin_ref[...]
    y_ref[...] = lax.dot_general(x2, wp1_ref[...], _DOT_T,
                                 preferred_element_type=jnp.float32)
    baseh_ref[...] = 0.5 * (
        lax.dot_general(x2, wp0_ref[...], _DOT_T,
                        preferred_element_type=jnp.float32) + btot_ref[...])
    wc0 = lax.dot_general(wp2_ref[...], we3_ref[...],
                          (((1,), (0,)), ((), ())),
                          preferred_element_type=jnp.float32)
    wc_ref[...] = wc0 * gamma_ref[...]
    c2_ref[...] = (lax.dot_general(be3_ref[...], wp2_ref[...], _DOT_T,
                                   preferred_element_type=jnp.float32)
                   + lax.dot_general(beta_ref[...], wc0, _DOT_T,
                                     preferred_element_type=jnp.float32))


SB = 80
EB = 4000


def _edge_body(seg, step0, nstep, ea_hbm, we0_ref, be0_ref, wc_ref, c2_ref,
               z_ref, ea_v, sems):
    i = pl.program_id(0)

    def blk_copy(step, slot):
        return pltpu.make_async_copy(
            ea_hbm.at[pl.ds((step0 + step) * EB, EB)], ea_v.at[slot],
            sems.at[slot])

    @pl.when(i == 0)
    def _():
        blk_copy(0, 0).start()

    @pl.when(i + 1 < nstep)
    def _():
        blk_copy(i + 1, (i + 1) % 2).start()
    blk_copy(i, i % 2).wait()
    a_all = lax.dot_general(ea_v[i % 2], we0_ref[...], _DOT_T,
                            preferred_element_type=jnp.float32)
    for s in range(EB // SB):
        a = a_all[s * SB:(s + 1) * SB, :] + be0_ref[...]
        e0 = jnp.maximum(a, 0.0)
        mu = jnp.mean(e0, axis=-1, keepdims=True)
        m2 = jnp.mean(e0 * e0, axis=-1, keepdims=True)
        rs = lax.rsqrt(m2 - mu * mu + 1e-5)
        nh = (e0 - mu) * rs
        z_ref[s * SB:(s + 1) * SB, :] = lax.dot_general(
            nh.astype(jnp.bfloat16), wc_ref[...], _DOT_T,
            preferred_element_type=jnp.float32) + c2_ref[...]


def _sc_gather_body(y_hbm, ei_hbm, baseh_hbm, out_hbm,
                    src_v, dst_v, row_v, acc, sems, dsems):
    cid = lax.axis_index("c")
    sid = lax.axis_index("s")
    wid = cid * NS + sid
    row0 = sid * ROWS_PER_TILE
    pltpu.sync_copy(baseh_hbm.at[pl.ds(row0, ROWS_PER_TILE)],
                    acc.at[pl.ds(row0, ROWS_PER_TILE)])
    edge0 = wid * EW
    pltpu.sync_copy(ei_hbm.at[pl.ds(edge0, EW)], src_v)
    plsc.subcore_barrier()

    pltpu.async_copy(ei_hbm.at[pl.ds(E + edge0, CE)], dst_v.at[0], dsems[0])
    pltpu.async_copy(y_hbm.at[src_v.at[pl.ds(0, CE)]], row_v.at[0], sems[0])

    def pair(p, carry):
        for b in range(2):
            k = 2 * p + b
            nb = 1 - b

            @pl.when(k < NCHUNK)
            def _():
                @pl.when(k + 1 < NCHUNK)
                def _():
                    pltpu.async_copy(
                        ei_hbm.at[pl.ds(E + edge0 + (k + 1) * CE, CE)],
                        dst_v.at[nb], dsems[nb])
                    pltpu.async_copy(
                        y_hbm.at[src_v.at[pl.ds((k + 1) * CE, CE)]],
                        row_v.at[nb], sems[nb])
                pltpu.make_async_copy(y_hbm.at[src_v.at[pl.ds(k * CE, CE)]],
                                      row_v.at[b], sems[b]).wait()
                pltpu.make_async_copy(
                    ei_hbm.at[pl.ds(E + edge0 + k * CE, CE)],
                    dst_v.at[b], dsems[b]).wait()
                pltpu.sync_copy(row_v.at[b], acc.at[dst_v.at[b]], add=True)
        return carry

    lax.fori_loop(0, NPAIR, pair, 0)
    plsc.subcore_barrier()
    pltpu.sync_copy(acc.at[pl.ds(row0, ROWS_PER_TILE)],
                    out_hbm.at[cid, pl.ds(row0, ROWS_PER_TILE)])


def _sc_scatter_body(seg, z_hbm, ei_hbm, part_hbm, out_hbm,
                     dst_v, row_v, acc, sems, dsems):
    cid = lax.axis_index("c")
    sid = lax.axis_index("s")
    wid = cid * NS + sid
    row0 = sid * ROWS_PER_TILE
    pltpu.sync_copy(part_hbm.at[cid, pl.ds(row0, ROWS_PER_TILE)],
                    acc.at[pl.ds(row0, ROWS_PER_TILE)])
    plsc.subcore_barrier()

    zrow0 = wid * EW2
    idx0 = E + seg * (E // 2) + zrow0

    for j in range(3):
        pltpu.async_copy(ei_hbm.at[pl.ds(idx0 + j * CE2, CE2)],
                         dst_v.at[j], dsems[j])
        pltpu.async_copy(z_hbm.at[pl.ds(zrow0 + j * CE2, CE2)],
                         row_v.at[j], sems[j])

    def quad(p, carry):
        for b in range(4):
            k = 4 * p + b
            nb = (b + 3) % 4

            @pl.when(k < NCHUNK2)
            def _():
                @pl.when(k + 3 < NCHUNK2)
                def _():
                    pltpu.async_copy(
                        ei_hbm.at[pl.ds(idx0 + (k + 3) * CE2, CE2)],
                        dst_v.at[nb], dsems[nb])
                    pltpu.async_copy(
                        z_hbm.at[pl.ds(zrow0 + (k + 3) * CE2, CE2)],
                        row_v.at[nb], sems[nb])
                pltpu.make_async_copy(z_hbm.at[pl.ds(zrow0 + k * CE2, CE2)],
                                      row_v.at[b], sems[b]).wait()
                pltpu.make_async_copy(
                    ei_hbm.at[pl.ds(idx0 + k * CE2, CE2)],
                    dst_v.at[b], dsems[b]).wait()
                pltpu.sync_copy(row_v.at[b], acc.at[dst_v.at[b]], add=True)
        return carry

    lax.fori_loop(0, (NCHUNK2 + 3) // 4, quad, 0)
    plsc.subcore_barrier()
    pltpu.sync_copy(acc.at[pl.ds(row0, ROWS_PER_TILE)],
                    out_hbm.at[cid, pl.ds(row0, ROWS_PER_TILE)])


def _combine_body(a_ref, b_ref, o_ref):
    o_ref[...] = a_ref[...] + b_ref[...]


def kernel(x, edge_index, edge_attr, W_lin, b_lin, W_e0, b_e0, ln_gamma,
           ln_beta, W_e3, b_e3, W_proj, b_proj, bias_p):
    ei = edge_index.astype(jnp.int32).reshape(2 * E)
    Wp0 = W_proj[:, 0:D]
    Wp1 = W_proj[:, D:2 * D]
    Wp2 = W_proj[:, 2 * D:3 * D]
    b_tot = (b_proj + bias_p).reshape(1, D)

    y, base_half, Wc, c2 = pl.pallas_call(
        _node_body,
        out_shape=(
            jax.ShapeDtypeStruct((N, D), jnp.float32),
            jax.ShapeDtypeStruct((N, D), jnp.float32),
            jax.ShapeDtypeStruct((D, D), jnp.float32),
            jax.ShapeDtypeStruct((1, D), jnp.float32),
        ),
    )(x, W_lin, b_lin.reshape(1, D), Wp0, Wp1, b_tot, Wp2, W_e3,
      b_e3.reshape(1, D), ln_gamma.reshape(1, D), ln_beta.reshape(1, D))

    NSTEP_HALF = E // 2 // EB
    Wc_bf = Wc.astype(jnp.bfloat16)

    def edge_half(seg):
        return pl.pallas_call(
            functools.partial(_edge_body, seg, seg * NSTEP_HALF, NSTEP_HALF),
            grid=(NSTEP_HALF,),
            in_specs=[
                pl.BlockSpec(memory_space=pltpu.MemorySpace.HBM),
                pl.BlockSpec((D, DE), lambda i: (0, 0)),
                pl.BlockSpec((1, D), lambda i: (0, 0)),
                pl.BlockSpec((D, D), lambda i: (0, 0)),
                pl.BlockSpec((1, D), lambda i: (0, 0)),
            ],
            out_specs=pl.BlockSpec((EB, D), lambda i: (i, 0)),
            out_shape=jax.ShapeDtypeStruct((E // 2, D), jnp.float32),
            scratch_shapes=[
                pltpu.VMEM((2, EB, DE), jnp.float32),
                pltpu.SemaphoreType.DMA((2,)),
            ],
        )(edge_attr, W_e0, b_e0.reshape(1, D), Wc_bf, c2)

    z_a = edge_half(0)
    z_b = edge_half(1)

    base_half_pad = jnp.pad(base_half, ((0, N_PAD - N), (0, 0)))
    mesh = plsc.VectorSubcoreMesh(core_axis_name="c", subcore_axis_name="s",
                                  num_cores=NC, num_subcores=NS)
    part1 = pl.kernel(
        _sc_gather_body,
        out_type=jax.ShapeDtypeStruct((NC, N_PAD, D), jnp.float32),
        mesh=mesh,
        scratch_types=[
            pltpu.VMEM((EW,), jnp.int32),
            pltpu.VMEM((2, CE), jnp.int32),
            pltpu.VMEM((2, CE, D), jnp.float32),
            pltpu.VMEM_SHARED((N_PAD, D), jnp.float32),
            (pltpu.SemaphoreType.DMA, pltpu.SemaphoreType.DMA),
            (pltpu.SemaphoreType.DMA, pltpu.SemaphoreType.DMA),
        ],
    )(y, ei, base_half_pad)

    def scatter_half(seg, zseg, part):
        return pl.kernel(
            functools.partial(_sc_scatter_body, seg),
            out_type=jax.ShapeDtypeStruct((NC, N_PAD, D), jnp.float32),
            mesh=mesh,
            scratch_types=[
                pltpu.VMEM((4, CE2), jnp.int32),
                pltpu.VMEM((4, CE2, D), jnp.float32),
                pltpu.VMEM_SHARED((N_PAD, D), jnp.float32),
                (pltpu.SemaphoreType.DMA,) * 4,
                (pltpu.SemaphoreType.DMA,) * 4,
            ],
        )(zseg, ei, part)

    part2 = scatter_half(0, z_a, part1)
    partials = scatter_half(1, z_b, part2)

    NB = 1000
    h = pl.pallas_call(
        _combine_body,
        grid=(N // NB,),
        in_specs=[
            pl.BlockSpec((NB, D), lambda i: (i, 0)),
            pl.BlockSpec((NB, D), lambda i: (i, 0)),
        ],
        out_specs=pl.BlockSpec((NB, D), lambda i: (i, 0)),
        out_shape=jax.ShapeDtypeStruct((N, D), jnp.float32),
    )(partials[0], partials[1])
    return h
